# baseline (speedup 1.0000x reference)
import functools
import math

import jax
import jax.numpy as jnp
import numpy as np
from jax import lax
from jax.experimental import pallas as pl
from jax.experimental.pallas import tpu as pltpu
from jax.experimental.pallas import tpu_sc as plsc

_ND, _RES, _NC, _NB = 64, 64, 4, 2
_N = 64 * 64 * 64
_RADIUS = math.sqrt(3.0)
_SCALE = (_RES - 1) / (2.0 * _RADIUS)

_NW = 32
_NCH = 6
_PR = 72
_PB = _NCH * _PR
_HROW = 4160
_HSZ = _NCH * _HROW + 256


def _directions_np():
    i = np.arange(_ND, dtype=np.float32)
    phi = np.float32((1 + 5**0.5) / 2)
    theta = np.float32(2 * math.pi) * i / phi
    z = (1 - 2 * (i + np.float32(0.5)) / np.float32(_ND)).astype(np.float32)
    r = np.sqrt(np.clip(1 - z * z, 0, None)).astype(np.float32)
    return np.stack([r * np.cos(theta), r * np.sin(theta), z], 0).astype(np.float32)


@functools.lru_cache(maxsize=1)
def _build_tables():
    dirs = _directions_np()
    amin = np.abs(dirs).argmin(axis=0)
    ax = np.linspace(-1.0, 1.0, 64, dtype=np.float32)
    groups = [np.where(amin == a)[0] for a in range(3)]
    tabs = []
    for a in range(3):
        oth = [u for u in range(3) if u != a]
        per_worker = [[] for _ in range(_NW)]
        for dl, d in enumerate(groups[a]):
            dv = dirs[:, d]
            u = (ax * dv[oth[0]])[:, None] + (ax * dv[oth[1]])[None, :]
            line = ax * dv[a]
            h = (u[:, :, None] + line[None, None, :]).astype(np.float32)
            b = np.clip(np.rint((h + np.float32(_RADIUS)) * np.float32(_SCALE)),
                        0, 63).astype(np.int64).reshape(4096, 64)
            change = np.ones((4096, 64), dtype=bool)
            change[:, 1:] = b[:, 1:] != b[:, :-1]
            colg, lstart = np.nonzero(change)
            lend = np.empty_like(lstart)
            lend[:-1] = lstart[1:] - 1
            lend[-1] = 63
            newcol = np.empty(len(colg), dtype=bool)
            newcol[:-1] = colg[1:] != colg[:-1]
            newcol[-1] = True
            lend[newcol] = 63
            rval = b[colg, lstart]
            dest = dl * 64 + rval
            packed = ((lend + 1).astype(np.uint32) << 25) | \
                     (lstart.astype(np.uint32) << 18) | \
                     ((colg & 127).astype(np.uint32) << 11) | dest.astype(np.uint32)
            wk = colg >> 7
            for w in range(_NW):
                sel = wk == w
                per_worker[w].append((packed[sel], dest[sel]))
        worker_tabs = []
        for w in range(_NW):
            packed = np.concatenate([p for p, _ in per_worker[w]])
            dest = np.concatenate([q for _, q in per_worker[w]])
            o = np.argsort(dest, kind="stable")
            sd = dest[o]
            first = np.r_[True, sd[1:] != sd[:-1]]
            idxs = np.arange(len(sd))
            start = np.maximum.accumulate(np.where(first, idxs, 0))
            seq = np.empty_like(idxs)
            seq[o] = idxs - start
            order = np.lexsort((dest, seq))
            packed = packed[order]
            seq = seq[order]
            colres = (packed >> 11) & 15
            o2 = np.lexsort((colres, seq))
            sr, rr = seq[o2], colres[o2]
            newgrp = np.r_[True, (sr[1:] != sr[:-1]) | (rr[1:] != rr[:-1])]
            idx2 = np.arange(len(sr))
            gstart = np.maximum.accumulate(np.where(newgrp, idx2, 0))
            rank = idx2 - gstart
            packed2 = packed[o2]
            o3 = np.lexsort((rr, rank, sr))
            packed = packed2[o3]
            seq = sr[o3]
            _, cnt = np.unique(seq, return_counts=True)
            pieces = []
            pos = 0
            for c in cnt:
                pieces.append(packed[pos:pos + c])
                pos += c
                pad = (-c) % 16
                if pad:
                    pieces.append((1472 + np.arange(pad, dtype=np.uint32)))
            worker_tabs.append(np.concatenate(pieces))
        maxb = max(len(t) for t in worker_tabs)
        maxb = ((maxb + 63) // 64) * 64
        tab = np.zeros((_NW, maxb), dtype=np.uint32)
        for w, t in enumerate(worker_tabs):
            tab[w, :len(t)] = t
            npad = maxb - len(t)
            if npad:
                tab[w, len(t):] = 1472 + (np.arange(npad, dtype=np.uint32) % 16)
        tabs.append(tab.view(np.int32))
    dcounts = [len(g) for g in groups]
    sizes = tuple(t.shape[1] for t in tabs)
    common = max(sizes)
    tabs = [np.pad(t, ((0, 0), (0, common - t.shape[1]))) for t in tabs]
    return tabs, sizes, dcounts


def _v_kernel(l_ref, t_ref, o_ref):
    l = l_ref[...]
    t = t_ref[...]
    cls3 = lax.broadcasted_iota(jnp.int32, (3, l.shape[1]), 0)
    outs = []
    for b in range(_NB):
        lb = l[4 * b:4 * b + 4]
        m = jnp.max(lb, axis=0, keepdims=True)
        e = jnp.exp(lb - m)
        pb = e * (1.0 / jnp.sum(e, axis=0, keepdims=True))
        oh = (t[b:b + 1] == cls3).astype(jnp.float32)
        outs.append(pb[0:3] - oh)
    o_ref[...] = jnp.concatenate(outs, axis=0).astype(jnp.bfloat16)


def _prefix_kernel(v0_ref, v1_ref, v2_ref, t_ref, p0_ref, p1_ref, p2_ref):
    t72 = t_ref[...]
    mm0 = jnp.dot(t72, v0_ref[0], preferred_element_type=jnp.float32)
    mm2 = lax.dot_general(t72, v2_ref[0], (((1,), (1,)), ((), ())),
                          preferred_element_type=jnp.float32)
    for w in range(_NW):
        p0_ref[w * _PR:(w + 1) * _PR, :] = mm0[:, w * 128:(w + 1) * 128]
        p2_ref[w * _PR:(w + 1) * _PR, :] = mm2[:, w * 128:(w + 1) * 128]
        vc = jnp.concatenate([v1_ref[0, 2 * w], v1_ref[0, 2 * w + 1]], axis=1)
        p1_ref[w * _PR:(w + 1) * _PR, :] = jnp.dot(
            t72, vc, preferred_element_type=jnp.float32)


def _sc_body(sizes, goffs, p0, p1, p2, t0, t1, t2, z, out,
             pb0, pb1, pb2, pb3, pb4, pb5, tbuf, hist):
    cid = lax.axis_index("c")
    sid = lax.axis_index("s")
    wid = sid * 2 + cid
    pbufs = (pb0, pb1, pb2, pb3, pb4, pb5)
    pltpu.sync_copy(z, hist)
    for g, (phbm, thbm) in enumerate(((p0, t0), (p1, t1), (p2, t2))):
        goff64 = goffs[g] * 64
        for c in range(_NCH):
            pltpu.sync_copy(
                phbm.at[pl.ds(c * (_NW * _PR) + wid * _PR, _PR)], pbufs[c])
        pltpu.sync_copy(thbm.at[wid], tbuf)

        @plsc.parallel_loop(0, sizes[g] // 16, unroll=4)
        def body(t, goff64=goff64):
            x = tbuf[pl.ds(t * 16, 16)]
            m_a = lax.shift_right_logical(x, 25)
            m_p = lax.shift_right_logical(x, 18) & 127
            col = lax.shift_right_logical(x, 11) & 127
            dst = (x & 2047) + goff64
            for c in range(_NCH):
                ga = plsc.load_gather(pbufs[c], [m_a, col])
                gp = plsc.load_gather(pbufs[c], [m_p, col])
                plsc.addupdate_scatter(hist, [dst + c * _HROW], ga - gp)

    pltpu.sync_copy(hist, out.at[wid])


def _loss_kernel(h_ref, u64_ref, o_ref):
    s = jnp.sum(h_ref[...], axis=0)
    ect = jnp.dot(s, u64_ref[...], preferred_element_type=jnp.float32)
    e0 = ect[0:65] + ect[65:130] + ect[130:195]
    e1 = ect[195:260] + ect[260:325] + ect[325:390]
    val = (jnp.sum(ect * ect) + jnp.sum(e0 * e0) + jnp.sum(e1 * e1)) * (
        1.0 / (_NB * _NC * _ND * _RES) / (float(_N) * float(_N)))
    o_ref[...] = jnp.full((8, 128), val, jnp.float32)


def kernel(logits, targets):
    tabs, sizes, dcounts = _build_tables()
    goffs = (0, dcounts[0], dcounts[0] + dcounts[1])

    logits8 = logits.reshape(_NB * _NC, _N)
    tgt = targets.reshape(_NB, _N).astype(jnp.int32)

    v6 = pl.pallas_call(
        _v_kernel,
        grid=(64,),
        in_specs=[
            pl.BlockSpec((_NB * _NC, 4096), lambda i: (0, i)),
            pl.BlockSpec((_NB, 4096), lambda i: (0, i)),
        ],
        out_specs=pl.BlockSpec((_NCH, 4096), lambda i: (0, i)),
        out_shape=jax.ShapeDtypeStruct((_NCH, _N), jnp.bfloat16),
    )(logits8, tgt)

    tri = np.zeros((_PR, 64), np.float32)
    tri[:65, :] = np.tril(np.ones((65, 64), np.float32), -1)
    t72 = jnp.asarray(tri, jnp.bfloat16)

    v0 = v6.reshape(_NCH, 64, 4096)
    v1 = v6.reshape(_NCH, 64, 64, 64)
    v2 = v6.reshape(_NCH, 4096, 64)
    prows = _NW * _PB

    p0, p1, p2 = pl.pallas_call(
        _prefix_kernel,
        grid=(_NCH,),
        in_specs=[
            pl.BlockSpec((1, 64, 4096), lambda c: (c, 0, 0)),
            pl.BlockSpec((1, 64, 64, 64), lambda c: (c, 0, 0, 0)),
            pl.BlockSpec((1, 4096, 64), lambda c: (c, 0, 0)),
            pl.BlockSpec((_PR, 64), lambda c: (0, 0)),
        ],
        out_specs=[
            pl.BlockSpec((_NW * _PR, 128), lambda c: (c, 0)),
            pl.BlockSpec((_NW * _PR, 128), lambda c: (c, 0)),
            pl.BlockSpec((_NW * _PR, 128), lambda c: (c, 0)),
        ],
        out_shape=[
            jax.ShapeDtypeStruct((prows, 128), jnp.float32),
            jax.ShapeDtypeStruct((prows, 128), jnp.float32),
            jax.ShapeDtypeStruct((prows, 128), jnp.float32),
        ],
    )(v0, v1, v2, t72)

    jt = [jnp.asarray(t) for t in tabs]
    zeros = jnp.zeros((_HSZ,), jnp.float32)

    mesh = plsc.VectorSubcoreMesh(core_axis_name="c", subcore_axis_name="s",
                                  num_cores=2, num_subcores=16)
    hist = pl.kernel(
        functools.partial(_sc_body, sizes, goffs),
        out_type=jax.ShapeDtypeStruct((_NW, _HSZ), jnp.float32),
        mesh=mesh,
        compiler_params=pltpu.CompilerParams(needs_layout_passes=False),
        scratch_types=[pltpu.VMEM((_PR, 128), jnp.float32)] * _NCH + [
            pltpu.VMEM((tabs[0].shape[1],), jnp.int32),
            pltpu.VMEM((_HSZ,), jnp.float32),
        ],
    )(p0, p1, p2, jt[0], jt[1], jt[2], zeros)

    hist4 = hist.reshape(_NW, _HSZ // 64, 64)
    u64 = jnp.asarray(np.triu(np.ones((64, 64), np.float32)))
    loss = pl.pallas_call(
        _loss_kernel,
        grid=(1,),
        in_specs=[
            pl.BlockSpec((_NW, _HSZ // 64, 64), lambda g: (0, 0, 0)),
            pl.BlockSpec((64, 64), lambda g: (0, 0)),
        ],
        out_specs=pl.BlockSpec((8, 128), lambda g: (0, 0)),
        out_shape=jax.ShapeDtypeStruct((8, 128), jnp.float32),
    )(hist4, u64)
    return loss[0, 0]


# speedup vs baseline: 1.1154x; 1.1154x over previous
import functools
import math

import jax
import jax.numpy as jnp
import numpy as np
from jax import lax
from jax.experimental import pallas as pl
from jax.experimental.pallas import tpu as pltpu
from jax.experimental.pallas import tpu_sc as plsc

_ND, _RES, _NC, _NB = 64, 64, 4, 2
_N = 64 * 64 * 64
_RADIUS = math.sqrt(3.0)
_SCALE = (_RES - 1) / (2.0 * _RADIUS)

_NW = 32
_NCH = 6
_PR = 72
_PB = _NCH * _PR
_HROW = 4160
_HSZ = _NCH * _HROW + 256


def _directions_np():
    i = np.arange(_ND, dtype=np.float32)
    phi = np.float32((1 + 5**0.5) / 2)
    theta = np.float32(2 * math.pi) * i / phi
    z = (1 - 2 * (i + np.float32(0.5)) / np.float32(_ND)).astype(np.float32)
    r = np.sqrt(np.clip(1 - z * z, 0, None)).astype(np.float32)
    return np.stack([r * np.cos(theta), r * np.sin(theta), z], 0).astype(np.float32)


@functools.lru_cache(maxsize=1)
def _build_tables():
    dirs = _directions_np()
    amin = np.abs(dirs).argmin(axis=0)
    ax = np.linspace(-1.0, 1.0, 64, dtype=np.float32)
    groups = [np.where(amin == a)[0] for a in range(3)]
    tabs = []
    for a in range(3):
        oth = [u for u in range(3) if u != a]
        per_worker = [[] for _ in range(_NW)]
        for dl, d in enumerate(groups[a]):
            dv = dirs[:, d]
            u = (ax * dv[oth[0]])[:, None] + (ax * dv[oth[1]])[None, :]
            line = ax * dv[a]
            h = (u[:, :, None] + line[None, None, :]).astype(np.float32)
            b = np.clip(np.rint((h + np.float32(_RADIUS)) * np.float32(_SCALE)),
                        0, 63).astype(np.int64).reshape(4096, 64)
            change = np.ones((4096, 64), dtype=bool)
            change[:, 1:] = b[:, 1:] != b[:, :-1]
            colg, lstart = np.nonzero(change)
            lend = np.empty_like(lstart)
            lend[:-1] = lstart[1:] - 1
            lend[-1] = 63
            newcol = np.empty(len(colg), dtype=bool)
            newcol[:-1] = colg[1:] != colg[:-1]
            newcol[-1] = True
            lend[newcol] = 63
            rval = b[colg, lstart]
            dest = dl * 64 + rval
            packed = ((lend + 1).astype(np.uint32) << 25) | \
                     (lstart.astype(np.uint32) << 18) | \
                     ((colg & 127).astype(np.uint32) << 11) | dest.astype(np.uint32)
            wk = colg >> 7
            for w in range(_NW):
                sel = wk == w
                per_worker[w].append((packed[sel], dest[sel]))
        worker_tabs = []
        for w in range(_NW):
            packed = np.concatenate([p for p, _ in per_worker[w]])
            dest = np.concatenate([q for _, q in per_worker[w]])
            o = np.argsort(dest, kind="stable")
            sd = dest[o]
            first = np.r_[True, sd[1:] != sd[:-1]]
            idxs = np.arange(len(sd))
            start = np.maximum.accumulate(np.where(first, idxs, 0))
            seq = np.empty_like(idxs)
            seq[o] = idxs - start
            order = np.lexsort((dest, seq))
            packed = packed[order]
            seq = seq[order]
            colres = (packed >> 11) & 15
            o2 = np.lexsort((colres, seq))
            sr, rr = seq[o2], colres[o2]
            newgrp = np.r_[True, (sr[1:] != sr[:-1]) | (rr[1:] != rr[:-1])]
            idx2 = np.arange(len(sr))
            gstart = np.maximum.accumulate(np.where(newgrp, idx2, 0))
            rank = idx2 - gstart
            packed2 = packed[o2]
            o3 = np.lexsort((rr, rank, sr))
            packed = packed2[o3]
            seq = sr[o3]
            _, cnt = np.unique(seq, return_counts=True)
            pieces = []
            pos = 0
            for c in cnt:
                pieces.append(packed[pos:pos + c])
                pos += c
                pad = (-c) % 16
                if pad:
                    pieces.append((1472 + np.arange(pad, dtype=np.uint32)))
            worker_tabs.append(np.concatenate(pieces))
        maxb = max(len(t) for t in worker_tabs)
        maxb = ((maxb + 63) // 64) * 64
        tab = np.zeros((_NW, maxb), dtype=np.uint32)
        for w, t in enumerate(worker_tabs):
            tab[w, :len(t)] = t
            npad = maxb - len(t)
            if npad:
                tab[w, len(t):] = 1472 + (np.arange(npad, dtype=np.uint32) % 16)
        tabs.append(tab.view(np.int32))
    dcounts = [len(g) for g in groups]
    sizes = tuple(t.shape[1] for t in tabs)
    common = max(sizes)
    tabs = [np.pad(t, ((0, 0), (0, common - t.shape[1]))) for t in tabs]
    return tabs, sizes, dcounts


def _v_kernel(l_ref, t_ref, o_ref):
    l = l_ref[...]
    t = t_ref[...]
    cls3 = lax.broadcasted_iota(jnp.int32, (3, l.shape[1]), 0)
    outs = []
    for b in range(_NB):
        lb = l[4 * b:4 * b + 4]
        m = jnp.max(lb, axis=0, keepdims=True)
        e = jnp.exp(lb - m)
        pb = e * (1.0 / jnp.sum(e, axis=0, keepdims=True))
        oh = (t[b:b + 1] == cls3).astype(jnp.float32)
        outs.append(pb[0:3] - oh)
    o_ref[...] = jnp.concatenate(outs, axis=0).astype(jnp.bfloat16)


def _prefix_kernel(v0_ref, v1_ref, v2_ref, t_ref, p0_ref, p1_ref, p2_ref):
    t72 = t_ref[...]
    mm0 = jnp.dot(t72, v0_ref[0], preferred_element_type=jnp.float32)
    mm2 = lax.dot_general(t72, v2_ref[0], (((1,), (1,)), ((), ())),
                          preferred_element_type=jnp.float32)
    for w in range(_NW):
        p0_ref[w * _PR:(w + 1) * _PR, :] = mm0[:, w * 128:(w + 1) * 128]
        p2_ref[w * _PR:(w + 1) * _PR, :] = mm2[:, w * 128:(w + 1) * 128]
        vc = jnp.concatenate([v1_ref[0, 2 * w], v1_ref[0, 2 * w + 1]], axis=1)
        p1_ref[w * _PR:(w + 1) * _PR, :] = jnp.dot(
            t72, vc, preferred_element_type=jnp.float32)


def _sc_body(sizes, goffs, p0, p1, p2, t0, t1, t2, z, out,
             pb0, pb1, pb2, pb3, pb4, pb5, tbuf, hist, sem):
    cid = lax.axis_index("c")
    sid = lax.axis_index("s")
    wid = sid * 2 + cid
    pbufs = (pb0, pb1, pb2, pb3, pb4, pb5)
    zcp = pltpu.async_copy(z, hist, sem)
    zcp.wait()
    for g, (phbm, thbm) in enumerate(((p0, t0), (p1, t1), (p2, t2))):
        goff64 = goffs[g] * 64
        cps = [pltpu.async_copy(
            phbm.at[pl.ds(c * (_NW * _PR) + wid * _PR, _PR)], pbufs[c], sem)
            for c in range(_NCH)]
        cps.append(pltpu.async_copy(thbm.at[wid], tbuf, sem))
        for cp in cps:
            cp.wait()

        @plsc.parallel_loop(0, sizes[g] // 16, unroll=8)
        def body(t, goff64=goff64):
            x = tbuf[pl.ds(t * 16, 16)]
            m_a = lax.shift_right_logical(x, 25)
            m_p = lax.shift_right_logical(x, 18) & 127
            col = lax.shift_right_logical(x, 11) & 127
            dst = (x & 2047) + goff64
            for c in range(_NCH):
                ga = plsc.load_gather(pbufs[c], [m_a, col])
                gp = plsc.load_gather(pbufs[c], [m_p, col])
                plsc.addupdate_scatter(hist, [dst + c * _HROW], ga - gp)

    pltpu.sync_copy(hist, out.at[wid])


def _loss_kernel(h_ref, u64_ref, o_ref):
    s = jnp.sum(h_ref[...], axis=0)
    ect = jnp.dot(s, u64_ref[...], preferred_element_type=jnp.float32)
    e0 = ect[0:65] + ect[65:130] + ect[130:195]
    e1 = ect[195:260] + ect[260:325] + ect[325:390]
    val = (jnp.sum(ect * ect) + jnp.sum(e0 * e0) + jnp.sum(e1 * e1)) * (
        1.0 / (_NB * _NC * _ND * _RES) / (float(_N) * float(_N)))
    o_ref[...] = jnp.full((8, 128), val, jnp.float32)


def kernel(logits, targets):
    tabs, sizes, dcounts = _build_tables()
    goffs = (0, dcounts[0], dcounts[0] + dcounts[1])

    logits8 = logits.reshape(_NB * _NC, _N)
    tgt = targets.reshape(_NB, _N).astype(jnp.int32)

    v6 = pl.pallas_call(
        _v_kernel,
        grid=(16,),
        in_specs=[
            pl.BlockSpec((_NB * _NC, 16384), lambda i: (0, i)),
            pl.BlockSpec((_NB, 16384), lambda i: (0, i)),
        ],
        out_specs=pl.BlockSpec((_NCH, 16384), lambda i: (0, i)),
        out_shape=jax.ShapeDtypeStruct((_NCH, _N), jnp.bfloat16),
    )(logits8, tgt)

    tri = np.zeros((_PR, 64), np.float32)
    tri[:65, :] = np.tril(np.ones((65, 64), np.float32), -1)
    t72 = jnp.asarray(tri, jnp.bfloat16)

    v0 = v6.reshape(_NCH, 64, 4096)
    v1 = v6.reshape(_NCH, 64, 64, 64)
    v2 = v6.reshape(_NCH, 4096, 64)
    prows = _NW * _PB

    p0, p1, p2 = pl.pallas_call(
        _prefix_kernel,
        grid=(_NCH,),
        in_specs=[
            pl.BlockSpec((1, 64, 4096), lambda c: (c, 0, 0)),
            pl.BlockSpec((1, 64, 64, 64), lambda c: (c, 0, 0, 0)),
            pl.BlockSpec((1, 4096, 64), lambda c: (c, 0, 0)),
            pl.BlockSpec((_PR, 64), lambda c: (0, 0)),
        ],
        out_specs=[
            pl.BlockSpec((_NW * _PR, 128), lambda c: (c, 0)),
            pl.BlockSpec((_NW * _PR, 128), lambda c: (c, 0)),
            pl.BlockSpec((_NW * _PR, 128), lambda c: (c, 0)),
        ],
        out_shape=[
            jax.ShapeDtypeStruct((prows, 128), jnp.float32),
            jax.ShapeDtypeStruct((prows, 128), jnp.float32),
            jax.ShapeDtypeStruct((prows, 128), jnp.float32),
        ],
    )(v0, v1, v2, t72)

    jt = [jnp.asarray(t) for t in tabs]
    zeros = jnp.zeros((_HSZ,), jnp.float32)

    mesh = plsc.VectorSubcoreMesh(core_axis_name="c", subcore_axis_name="s",
                                  num_cores=2, num_subcores=16)
    hist = pl.kernel(
        functools.partial(_sc_body, sizes, goffs),
        out_type=jax.ShapeDtypeStruct((_NW, _HSZ), jnp.float32),
        mesh=mesh,
        compiler_params=pltpu.CompilerParams(needs_layout_passes=False),
        scratch_types=[pltpu.VMEM((_PR, 128), jnp.float32)] * _NCH + [
            pltpu.VMEM((tabs[0].shape[1],), jnp.int32),
            pltpu.VMEM((_HSZ,), jnp.float32),
            pltpu.SemaphoreType.DMA,
        ],
    )(p0, p1, p2, jt[0], jt[1], jt[2], zeros)

    hist4 = hist.reshape(_NW, _HSZ // 64, 64)
    u64 = jnp.asarray(np.triu(np.ones((64, 64), np.float32)))
    loss = pl.pallas_call(
        _loss_kernel,
        grid=(1,),
        in_specs=[
            pl.BlockSpec((_NW, _HSZ // 64, 64), lambda g: (0, 0, 0)),
            pl.BlockSpec((64, 64), lambda g: (0, 0)),
        ],
        out_specs=pl.BlockSpec((8, 128), lambda g: (0, 0)),
        out_shape=jax.ShapeDtypeStruct((8, 128), jnp.float32),
    )(hist4, u64)
    return loss[0, 0]
